# Initial kernel scaffold; baseline (speedup 1.0000x reference)
import jax
import jax.numpy as jnp
from jax.experimental import pallas as pl


def kernel(xyz, points, conv_w0, conv_b0, bn_g0, bn_b0, conv_w1, conv_b1, bn_g1, bn_b1, conv_w2, conv_b2, bn_g2, bn_b2):
    raise NotImplementedError("write your pallas kernel here")


# speedup vs baseline: 11.8640x; 11.8640x over previous
import functools

import jax
import jax.numpy as jnp
from jax import lax
from jax.experimental import pallas as pl
from jax.experimental.pallas import tpu as pltpu
from jax.experimental.pallas import tpu_sc as plsc

_B, _N, _CF = 8, 4096, 64
_S = 1024
_K = 32
_R2 = 0.5 ** 2
_M = _B * _S * _K
_CPAD = 128
_TILE_M = 1024
_STILE = 128


def _fps_body(x_ref, y_ref, z_ref, newc_ref):
    x = x_ref[...]
    y = y_ref[...]
    z = z_ref[...]
    col = lax.broadcasted_iota(jnp.int32, (_B, _N), 1)

    def body(i, state):
        distance, farthest = state
        onehot = col == farthest
        cx = jnp.sum(jnp.where(onehot, x, 0.0), axis=1, keepdims=True)
        cy = jnp.sum(jnp.where(onehot, y, 0.0), axis=1, keepdims=True)
        cz = jnp.sum(jnp.where(onehot, z, 0.0), axis=1, keepdims=True)
        newc_ref[pl.ds(i, 1), :, :] = jnp.concatenate([cx, cy, cz], axis=1)[None]
        dx = x - cx
        dy = y - cy
        dz = z - cz
        d = (dx * dx + dy * dy) + dz * dz
        distance = jnp.minimum(distance, d)
        mx = jnp.max(distance, axis=1, keepdims=True)
        farthest = jnp.min(jnp.where(distance == mx, col, _N), axis=1,
                           keepdims=True)
        return distance, farthest

    dist0 = jnp.full((_B, _N), 1e10, dtype=jnp.float32)
    far0 = jnp.zeros((_B, 1), dtype=jnp.int32)
    lax.fori_loop(0, _S, body, (dist0, far0))


def _fps(x, y, z, interpret=False):
    return pl.pallas_call(
        _fps_body,
        out_shape=jax.ShapeDtypeStruct((_S, _B, 3), jnp.float32),
        interpret=interpret,
    )(x, y, z)


def _ballq_body(x_ref, y_ref, z_ref, q_ref, idx_ref):
    x = x_ref[0]
    y = y_ref[0]
    z = z_ref[0]
    q = q_ref[...]
    qx = q[:, 0:1]
    qy = q[:, 1:2]
    qz = q[:, 2:3]
    def _bf(v):
        return v.astype(jnp.bfloat16).astype(jnp.float32)
    inner = _bf(qx) * _bf(x) + _bf(qy) * _bf(y) + _bf(qz) * _bf(z)
    pn = x * x + y * y + z * z
    qn = qx * qx + qy * qy + qz * qz
    d2 = -2.0 * inner
    d2 = d2 + qn
    d2 = d2 + pn
    col = lax.broadcasted_iota(jnp.int32, (_STILE, _N), 1)
    cand = jnp.where(d2 > _R2, _N, col)
    cols = []
    for _ in range(_K):
        m = jnp.min(cand, axis=1, keepdims=True)
        cols.append(m)
        cand = jnp.where(cand == m, _N, cand)
    idx = jnp.concatenate(cols, axis=1)
    idx = jnp.where(idx == _N, cols[0], idx)
    idx_ref[...] = idx[None]


def _ballq(x, y, z, nxf, interpret=False):
    grid = (_B, _S // _STILE)
    return pl.pallas_call(
        _ballq_body,
        grid=grid,
        in_specs=[
            pl.BlockSpec((1, 1, _N), lambda b, j: (b, 0, 0)),
            pl.BlockSpec((1, 1, _N), lambda b, j: (b, 0, 0)),
            pl.BlockSpec((1, 1, _N), lambda b, j: (b, 0, 0)),
            pl.BlockSpec((_STILE, 3),
                         lambda b, j: (b * (_S // _STILE) + j, 0)),
        ],
        out_specs=pl.BlockSpec((1, _STILE, _K), lambda b, j: (b, j, 0)),
        out_shape=jax.ShapeDtypeStruct((_B, _S, _K), jnp.int32),
        interpret=interpret,
    )(x, y, z, nxf)


_CHUNK = 128


def _group_gather(table, flat_idx):
    info = plsc.get_sparse_core_info()
    nc, ns = info.num_cores, info.num_subcores
    nw = nc * ns
    rows_per_w = _M // nw
    n_chunks = rows_per_w // _CHUNK
    mesh = plsc.VectorSubcoreMesh(core_axis_name="c", subcore_axis_name="s")

    @functools.partial(
        pl.kernel,
        mesh=mesh,
        out_type=jax.ShapeDtypeStruct((_M, _CPAD), jnp.float32),
        scratch_types=[
            pltpu.VMEM((_CHUNK,), jnp.int32),
            pltpu.VMEM((_CHUNK, _CPAD), jnp.float32),
            pltpu.SemaphoreType.DMA,
        ],
    )
    def k(table_hbm, idx_hbm, out_hbm, idx_v, rows_v, sem):
        wid = lax.axis_index("s") * nc + lax.axis_index("c")
        w_base = wid * rows_per_w

        def body(c, carry):
            base = w_base + c * _CHUNK
            pltpu.sync_copy(idx_hbm.at[pl.ds(base, _CHUNK)], idx_v)
            pltpu.async_copy(table_hbm.at[idx_v], rows_v, sem).wait()
            pltpu.sync_copy(rows_v, out_hbm.at[pl.ds(base, _CHUNK)])
            return carry

        lax.fori_loop(0, n_chunks, body, 0)

    return k(table, flat_idx)


def _k0_body(g_ref, nx_ref, w0p_ref, wxyz_ref, b0_ref, a1_ref, st_ref):
    i = pl.program_id(0)
    g = g_ref[...]
    nx = nx_ref[...]
    bias = jnp.dot(nx, wxyz_ref[...], preferred_element_type=jnp.float32)
    biasf = jnp.broadcast_to(bias[:, None, :], (_TILE_M // _K, _K, 64))
    biasf = biasf.reshape(_TILE_M, 64)
    y = jnp.dot(g, w0p_ref[...], preferred_element_type=jnp.float32)
    y = y + b0_ref[...] - biasf
    a1_ref[...] = y

    @pl.when(i == 0)
    def _():
        st_ref[...] = jnp.zeros_like(st_ref)

    s = jnp.sum(y, axis=0, keepdims=True)
    ss = jnp.sum(y * y, axis=0, keepdims=True)
    st_ref[0:2, :] = st_ref[0:2, :] + jnp.concatenate([s, ss], axis=0)


def _k0(g, nxf, w0p, wxyz, b0r, interpret=False):
    grid = (_M // _TILE_M,)
    return pl.pallas_call(
        _k0_body,
        grid=grid,
        in_specs=[
            pl.BlockSpec((_TILE_M, _CPAD), lambda i: (i, 0)),
            pl.BlockSpec((_TILE_M // _K, 3), lambda i: (i, 0)),
            pl.BlockSpec((_CPAD, 64), lambda i: (0, 0)),
            pl.BlockSpec((3, 64), lambda i: (0, 0)),
            pl.BlockSpec((1, 64), lambda i: (0, 0)),
        ],
        out_specs=[
            pl.BlockSpec((_TILE_M, 64), lambda i: (i, 0)),
            pl.BlockSpec((8, 64), lambda i: (0, 0)),
        ],
        out_shape=[
            jax.ShapeDtypeStruct((_M, 64), jnp.float32),
            jax.ShapeDtypeStruct((8, 64), jnp.float32),
        ],
        interpret=interpret,
    )(g, nxf, w0p, wxyz, b0r)


def _bn_scale_shift(st_ref, g_r, beta_r):
    mean = st_ref[0:1, :] * (1.0 / _M)
    ex2 = st_ref[1:2, :] * (1.0 / _M)
    var = ex2 - mean * mean
    scale = g_r * lax.rsqrt(var + 1e-5)
    shift = beta_r - mean * scale
    return scale, shift


def _layer_body(a_ref, st_ref, g_ref, beta_ref, w_ref, b_ref, out_ref,
                stn_ref):
    i = pl.program_id(0)
    scale, shift = _bn_scale_shift(st_ref, g_ref[...], beta_ref[...])
    xin = jnp.maximum(a_ref[...] * scale + shift, 0.0)
    y = jnp.dot(xin, w_ref[...], preferred_element_type=jnp.float32)
    y = y + b_ref[...]
    out_ref[...] = y

    @pl.when(i == 0)
    def _():
        stn_ref[...] = jnp.zeros_like(stn_ref)

    s = jnp.sum(y, axis=0, keepdims=True)
    ss = jnp.sum(y * y, axis=0, keepdims=True)
    stn_ref[0:2, :] = stn_ref[0:2, :] + jnp.concatenate([s, ss], axis=0)


def _layer(a, st, g_r, beta_r, w_t, b_r, cin, cout, interpret=False):
    grid = (_M // _TILE_M,)
    return pl.pallas_call(
        _layer_body,
        grid=grid,
        in_specs=[
            pl.BlockSpec((_TILE_M, cin), lambda i: (i, 0)),
            pl.BlockSpec((8, cin), lambda i: (0, 0)),
            pl.BlockSpec((1, cin), lambda i: (0, 0)),
            pl.BlockSpec((1, cin), lambda i: (0, 0)),
            pl.BlockSpec((cin, cout), lambda i: (0, 0)),
            pl.BlockSpec((1, cout), lambda i: (0, 0)),
        ],
        out_specs=[
            pl.BlockSpec((_TILE_M, cout), lambda i: (i, 0)),
            pl.BlockSpec((8, cout), lambda i: (0, 0)),
        ],
        out_shape=[
            jax.ShapeDtypeStruct((_M, cout), jnp.float32),
            jax.ShapeDtypeStruct((8, cout), jnp.float32),
        ],
        interpret=interpret,
    )(a, st, g_r, beta_r, w_t, b_r)


def _k3_body(a_ref, st_ref, g_ref, beta_ref, out_ref):
    scale, shift = _bn_scale_shift(st_ref, g_ref[...], beta_ref[...])
    xin = jnp.maximum(a_ref[...] * scale + shift, 0.0)
    x3 = xin.reshape(_TILE_M // _K, _K, 256)
    out_ref[...] = jnp.max(x3, axis=1)


def _k3(a, st, g_r, beta_r, interpret=False):
    grid = (_M // _TILE_M,)
    return pl.pallas_call(
        _k3_body,
        grid=grid,
        in_specs=[
            pl.BlockSpec((_TILE_M, 256), lambda i: (i, 0)),
            pl.BlockSpec((8, 256), lambda i: (0, 0)),
            pl.BlockSpec((1, 256), lambda i: (0, 0)),
            pl.BlockSpec((1, 256), lambda i: (0, 0)),
        ],
        out_specs=pl.BlockSpec((_TILE_M // _K, 256), lambda i: (i, 0)),
        out_shape=jax.ShapeDtypeStruct((_B * _S, 256), jnp.float32),
        interpret=interpret,
    )(a, st, g_r, beta_r)


@jax.jit
def kernel(xyz, points, conv_w0, conv_b0, bn_g0, bn_b0, conv_w1, conv_b1,
           bn_g1, bn_b1, conv_w2, conv_b2, bn_g2, bn_b2):
    x = xyz[:, :, 0]
    y = xyz[:, :, 1]
    z = xyz[:, :, 2]

    newc = _fps(x, y, z)
    new_xyz = jnp.transpose(newc, (1, 0, 2))
    nxf = new_xyz.reshape(_B * _S, 3)

    idx = _ballq(x[:, None, :], y[:, None, :], z[:, None, :], nxf)
    flat_idx = (idx + (jnp.arange(_B, dtype=jnp.int32) * _N)[:, None, None])
    flat_idx = flat_idx.reshape(_M)

    table = jnp.concatenate([xyz, points], axis=-1).reshape(_B * _N, 67)
    table = jnp.pad(table, ((0, 0), (0, _CPAD - 67)))
    gathered = _group_gather(table, flat_idx)

    w0p = jnp.pad(conv_w0.T, ((0, _CPAD - 67), (0, 0)))
    wxyz = conv_w0[:, :3].T

    a1, st0 = _k0(gathered, nxf, w0p, wxyz, conv_b0[None, :])
    a2, st1 = _layer(a1, st0, bn_g0[None, :], bn_b0[None, :], conv_w1.T,
                     conv_b1[None, :], 64, 128)
    a3, st2 = _layer(a2, st1, bn_g1[None, :], bn_b1[None, :], conv_w2.T,
                     conv_b2[None, :], 128, 256)
    out = _k3(a3, st2, bn_g2[None, :], bn_b2[None, :])

    return new_xyz, out.reshape(_B, _S, 256)


# speedup vs baseline: 13.1372x; 1.1073x over previous
import functools

import jax
import jax.numpy as jnp
from jax import lax
from jax.experimental import pallas as pl
from jax.experimental.pallas import tpu as pltpu
from jax.experimental.pallas import tpu_sc as plsc

_B, _N, _CF = 8, 4096, 64
_S = 1024
_K = 32
_R2 = 0.5 ** 2
_M = _B * _S * _K
_CPAD = 128
_TILE_M = 1024
_STILE = 256


def _fps_body(x_ref, y_ref, z_ref, newc_ref):
    x = x_ref[...]
    y = y_ref[...]
    z = z_ref[...]
    col = lax.broadcasted_iota(jnp.int32, (_B, _N), 1)

    def body(i, state):
        distance, farthest = state
        onehot = col == farthest
        cx = jnp.sum(jnp.where(onehot, x, 0.0), axis=1, keepdims=True)
        cy = jnp.sum(jnp.where(onehot, y, 0.0), axis=1, keepdims=True)
        cz = jnp.sum(jnp.where(onehot, z, 0.0), axis=1, keepdims=True)
        newc_ref[pl.ds(i, 1), :, :] = jnp.concatenate([cx, cy, cz], axis=1)[None]
        dx = x - cx
        dy = y - cy
        dz = z - cz
        d = (dx * dx + dy * dy) + dz * dz
        distance = jnp.minimum(distance, d)
        mx = jnp.max(distance, axis=1, keepdims=True)
        farthest = jnp.min(jnp.where(distance == mx, col, _N), axis=1,
                           keepdims=True)
        return distance, farthest

    dist0 = jnp.full((_B, _N), 1e10, dtype=jnp.float32)
    far0 = jnp.zeros((_B, 1), dtype=jnp.int32)
    lax.fori_loop(0, _S, body, (dist0, far0))


def _fps(x, y, z, interpret=False):
    return pl.pallas_call(
        _fps_body,
        out_shape=jax.ShapeDtypeStruct((_S, _B, 3), jnp.float32),
        interpret=interpret,
    )(x, y, z)


def _ballq_body(x_ref, y_ref, z_ref, q_ref, idx_ref):
    x = x_ref[0]
    y = y_ref[0]
    z = z_ref[0]
    q = q_ref[...]
    qx = q[:, 0:1]
    qy = q[:, 1:2]
    qz = q[:, 2:3]
    def _bf(v):
        return v.astype(jnp.bfloat16).astype(jnp.float32)
    inner = _bf(qx) * _bf(x) + _bf(qy) * _bf(y) + _bf(qz) * _bf(z)
    pn = x * x + y * y + z * z
    qn = qx * qx + qy * qy + qz * qz
    d2 = -2.0 * inner
    d2 = d2 + qn
    d2 = d2 + pn
    col = lax.broadcasted_iota(jnp.int32, (_STILE, _N), 1)
    cand = jnp.where(d2 > _R2, _N, col)
    cols = []
    for _ in range(_K):
        m = jnp.min(cand, axis=1, keepdims=True)
        cols.append(m)
        cand = jnp.where(cand == m, _N, cand)
    idx = jnp.concatenate(cols, axis=1)
    idx = jnp.where(idx == _N, cols[0], idx)
    idx_ref[...] = idx[None]


def _ballq(x, y, z, nxf, interpret=False):
    grid = (_B, _S // _STILE)
    return pl.pallas_call(
        _ballq_body,
        grid=grid,
        in_specs=[
            pl.BlockSpec((1, 1, _N), lambda b, j: (b, 0, 0)),
            pl.BlockSpec((1, 1, _N), lambda b, j: (b, 0, 0)),
            pl.BlockSpec((1, 1, _N), lambda b, j: (b, 0, 0)),
            pl.BlockSpec((_STILE, 3),
                         lambda b, j: (b * (_S // _STILE) + j, 0)),
        ],
        out_specs=pl.BlockSpec((1, _STILE, _K), lambda b, j: (b, j, 0)),
        out_shape=jax.ShapeDtypeStruct((_B, _S, _K), jnp.int32),
        interpret=interpret,
    )(x, y, z, nxf)


_CHUNK = 128


def _group_gather(table, flat_idx):
    info = plsc.get_sparse_core_info()
    nc, ns = info.num_cores, info.num_subcores
    nw = nc * ns
    rows_per_w = _M // nw
    mesh = plsc.VectorSubcoreMesh(core_axis_name="c", subcore_axis_name="s")

    burst = 4
    rows_per_burst = burst * _CHUNK
    n_bursts = rows_per_w // rows_per_burst

    @functools.partial(
        pl.kernel,
        mesh=mesh,
        out_type=jax.ShapeDtypeStruct((_M, _CPAD), jnp.float32),
        scratch_types=[
            pltpu.VMEM((rows_per_w,), jnp.int32),
            pltpu.VMEM((rows_per_burst, _CPAD), jnp.float32),
            pltpu.SemaphoreType.DMA,
        ],
    )
    def k(table_hbm, idx_hbm, out_hbm, idx_v, rows_v, sem):
        wid = lax.axis_index("s") * nc + lax.axis_index("c")
        w_base = wid * rows_per_w
        pltpu.sync_copy(idx_hbm.at[pl.ds(w_base, rows_per_w)], idx_v)

        def body(o, carry):
            handles = []
            for j in range(burst):
                off = o * rows_per_burst + j * _CHUNK
                handles.append(pltpu.async_copy(
                    table_hbm.at[idx_v.at[pl.ds(off, _CHUNK)]],
                    rows_v.at[pl.ds(j * _CHUNK, _CHUNK)], sem))
            for h in handles:
                h.wait()
            pltpu.sync_copy(
                rows_v, out_hbm.at[pl.ds(w_base + o * rows_per_burst,
                                         rows_per_burst)])
            return carry

        lax.fori_loop(0, n_bursts, body, 0)

    return k(table, flat_idx)


def _k0_body(g_ref, nx_ref, w0p_ref, wxyz_ref, b0_ref, a1_ref, st_ref):
    i = pl.program_id(0)
    g = g_ref[...]
    nx = nx_ref[...]
    bias = jnp.dot(nx, wxyz_ref[...], preferred_element_type=jnp.float32)
    biasf = jnp.broadcast_to(bias[:, None, :], (_TILE_M // _K, _K, 64))
    biasf = biasf.reshape(_TILE_M, 64)
    y = jnp.dot(g, w0p_ref[...], preferred_element_type=jnp.float32)
    y = y + b0_ref[...] - biasf
    a1_ref[...] = y

    @pl.when(i == 0)
    def _():
        st_ref[...] = jnp.zeros_like(st_ref)

    s = jnp.sum(y, axis=0, keepdims=True)
    ss = jnp.sum(y * y, axis=0, keepdims=True)
    st_ref[0:2, :] = st_ref[0:2, :] + jnp.concatenate([s, ss], axis=0)


def _k0(g, nxf, w0p, wxyz, b0r, interpret=False):
    grid = (_M // _TILE_M,)
    return pl.pallas_call(
        _k0_body,
        grid=grid,
        in_specs=[
            pl.BlockSpec((_TILE_M, _CPAD), lambda i: (i, 0)),
            pl.BlockSpec((_TILE_M // _K, 3), lambda i: (i, 0)),
            pl.BlockSpec((_CPAD, 64), lambda i: (0, 0)),
            pl.BlockSpec((3, 64), lambda i: (0, 0)),
            pl.BlockSpec((1, 64), lambda i: (0, 0)),
        ],
        out_specs=[
            pl.BlockSpec((_TILE_M, 64), lambda i: (i, 0)),
            pl.BlockSpec((8, 64), lambda i: (0, 0)),
        ],
        out_shape=[
            jax.ShapeDtypeStruct((_M, 64), jnp.float32),
            jax.ShapeDtypeStruct((8, 64), jnp.float32),
        ],
        interpret=interpret,
    )(g, nxf, w0p, wxyz, b0r)


def _bn_scale_shift(st_ref, g_r, beta_r):
    mean = st_ref[0:1, :] * (1.0 / _M)
    ex2 = st_ref[1:2, :] * (1.0 / _M)
    var = ex2 - mean * mean
    scale = g_r * lax.rsqrt(var + 1e-5)
    shift = beta_r - mean * scale
    return scale, shift


def _gram_scale_shift(gram, cs_row, w_t, b_r, g_r, beta_r):
    inner_cs = jnp.dot(cs_row, w_t, preferred_element_type=jnp.float32)
    t = jnp.dot(gram, w_t, preferred_element_type=jnp.float32)
    diag = jnp.sum(w_t * t, axis=0, keepdims=True)
    ssum = inner_cs + _M * b_r
    ssq = diag + 2.0 * b_r * inner_cs + _M * b_r * b_r
    mean = ssum * (1.0 / _M)
    var = ssq * (1.0 / _M) - mean * mean
    scale = g_r * lax.rsqrt(var + 1e-5)
    shift = beta_r - mean * scale
    return scale, shift


def _accum_gram(i, x, gram_ref, cs_ref):
    @pl.when(i == 0)
    def _():
        gram_ref[...] = jnp.zeros_like(gram_ref)
        cs_ref[...] = jnp.zeros_like(cs_ref)

    g = lax.dot_general(x, x, (((0,), (0,)), ((), ())),
                        preferred_element_type=jnp.float32)
    gram_ref[...] = gram_ref[...] + g
    cs_ref[0:1, :] = cs_ref[0:1, :] + jnp.sum(x, axis=0, keepdims=True)


def _kx1_body(a_ref, st_ref, g_ref, beta_ref, x_ref, gram_ref, cs_ref):
    i = pl.program_id(0)
    scale, shift = _bn_scale_shift(st_ref, g_ref[...], beta_ref[...])
    x = jnp.maximum(a_ref[...] * scale + shift, 0.0)
    x_ref[...] = x
    _accum_gram(i, x, gram_ref, cs_ref)


def _kx1(a, st, g_r, beta_r, interpret=False):
    grid = (_M // _TILE_M,)
    return pl.pallas_call(
        _kx1_body,
        grid=grid,
        in_specs=[
            pl.BlockSpec((_TILE_M, 64), lambda i: (i, 0)),
            pl.BlockSpec((8, 64), lambda i: (0, 0)),
            pl.BlockSpec((1, 64), lambda i: (0, 0)),
            pl.BlockSpec((1, 64), lambda i: (0, 0)),
        ],
        out_specs=[
            pl.BlockSpec((_TILE_M, 64), lambda i: (i, 0)),
            pl.BlockSpec((64, 64), lambda i: (0, 0)),
            pl.BlockSpec((8, 64), lambda i: (0, 0)),
        ],
        out_shape=[
            jax.ShapeDtypeStruct((_M, 64), jnp.float32),
            jax.ShapeDtypeStruct((64, 64), jnp.float32),
            jax.ShapeDtypeStruct((8, 64), jnp.float32),
        ],
        interpret=interpret,
    )(a, st, g_r, beta_r)


def _kx2_body(x_ref, gram_ref, cs_ref, g_ref, beta_ref, w_ref, b_ref,
              x2_ref, gram2_ref, cs2_ref):
    i = pl.program_id(0)
    w_t = w_ref[...]
    b_r = b_ref[...]
    scale, shift = _gram_scale_shift(gram_ref[...], cs_ref[0:1, :], w_t, b_r,
                                     g_ref[...], beta_ref[...])
    y = jnp.dot(x_ref[...], w_t, preferred_element_type=jnp.float32) + b_r
    x2 = jnp.maximum(y * scale + shift, 0.0)
    x2_ref[...] = x2
    _accum_gram(i, x2, gram2_ref, cs2_ref)


def _kx2(x1, gram1, cs1, g_r, beta_r, w_t, b_r, cin, cout, interpret=False):
    grid = (_M // _TILE_M,)
    return pl.pallas_call(
        _kx2_body,
        grid=grid,
        in_specs=[
            pl.BlockSpec((_TILE_M, cin), lambda i: (i, 0)),
            pl.BlockSpec((cin, cin), lambda i: (0, 0)),
            pl.BlockSpec((8, cin), lambda i: (0, 0)),
            pl.BlockSpec((1, cout), lambda i: (0, 0)),
            pl.BlockSpec((1, cout), lambda i: (0, 0)),
            pl.BlockSpec((cin, cout), lambda i: (0, 0)),
            pl.BlockSpec((1, cout), lambda i: (0, 0)),
        ],
        out_specs=[
            pl.BlockSpec((_TILE_M, cout), lambda i: (i, 0)),
            pl.BlockSpec((cout, cout), lambda i: (0, 0)),
            pl.BlockSpec((8, cout), lambda i: (0, 0)),
        ],
        out_shape=[
            jax.ShapeDtypeStruct((_M, cout), jnp.float32),
            jax.ShapeDtypeStruct((cout, cout), jnp.float32),
            jax.ShapeDtypeStruct((8, cout), jnp.float32),
        ],
        interpret=interpret,
    )(x1, gram1, cs1, g_r, beta_r, w_t, b_r)


def _kx3_body(x_ref, gram_ref, cs_ref, g_ref, beta_ref, w_ref, b_ref,
              out_ref):
    w_t = w_ref[...]
    b_r = b_ref[...]
    scale, shift = _gram_scale_shift(gram_ref[...], cs_ref[0:1, :], w_t, b_r,
                                     g_ref[...], beta_ref[...])
    y = jnp.dot(x_ref[...], w_t, preferred_element_type=jnp.float32) + b_r
    x3 = jnp.maximum(y * scale + shift, 0.0)
    x3 = x3.reshape(_TILE_M // _K, _K, 256)
    out_ref[...] = jnp.max(x3, axis=1)


def _kx3(x2, gram2, cs2, g_r, beta_r, w_t, b_r, interpret=False):
    grid = (_M // _TILE_M,)
    return pl.pallas_call(
        _kx3_body,
        grid=grid,
        in_specs=[
            pl.BlockSpec((_TILE_M, 128), lambda i: (i, 0)),
            pl.BlockSpec((128, 128), lambda i: (0, 0)),
            pl.BlockSpec((8, 128), lambda i: (0, 0)),
            pl.BlockSpec((1, 256), lambda i: (0, 0)),
            pl.BlockSpec((1, 256), lambda i: (0, 0)),
            pl.BlockSpec((128, 256), lambda i: (0, 0)),
            pl.BlockSpec((1, 256), lambda i: (0, 0)),
        ],
        out_specs=pl.BlockSpec((_TILE_M // _K, 256), lambda i: (i, 0)),
        out_shape=jax.ShapeDtypeStruct((_B * _S, 256), jnp.float32),
        interpret=interpret,
    )(x2, gram2, cs2, g_r, beta_r, w_t, b_r)


@jax.jit
def kernel(xyz, points, conv_w0, conv_b0, bn_g0, bn_b0, conv_w1, conv_b1,
           bn_g1, bn_b1, conv_w2, conv_b2, bn_g2, bn_b2):
    x = xyz[:, :, 0]
    y = xyz[:, :, 1]
    z = xyz[:, :, 2]

    newc = _fps(x, y, z)
    new_xyz = jnp.transpose(newc, (1, 0, 2))
    nxf = new_xyz.reshape(_B * _S, 3)

    idx = _ballq(x[:, None, :], y[:, None, :], z[:, None, :], nxf)
    flat_idx = (idx + (jnp.arange(_B, dtype=jnp.int32) * _N)[:, None, None])
    flat_idx = flat_idx.reshape(_M)

    table = jnp.concatenate([xyz, points], axis=-1).reshape(_B * _N, 67)
    table = jnp.pad(table, ((0, 0), (0, _CPAD - 67)))
    gathered = _group_gather(table, flat_idx)

    w0p = jnp.pad(conv_w0.T, ((0, _CPAD - 67), (0, 0)))
    wxyz = conv_w0[:, :3].T

    a1, st0 = _k0(gathered, nxf, w0p, wxyz, conv_b0[None, :])
    x1, gram1, cs1 = _kx1(a1, st0, bn_g0[None, :], bn_b0[None, :])
    x2, gram2, cs2 = _kx2(x1, gram1, cs1, bn_g1[None, :], bn_b1[None, :],
                          conv_w1.T, conv_b1[None, :], 64, 128)
    out = _kx3(x2, gram2, cs2, bn_g2[None, :], bn_b2[None, :],
               conv_w2.T, conv_b2[None, :])

    return new_xyz, out.reshape(_B, _S, 256)
